# baseline (speedup 1.0000x reference)
import functools

import jax
import jax.numpy as jnp
from jax import lax
from jax.experimental import pallas as pl
from jax.experimental.pallas import tpu as pltpu
from jax.experimental.pallas import tpu_sc as plsc

BATCH = 4096
SEQ = 200
HIDDEN = 64
VOCAB = 1000000
LANES = 128

NC = 2
NS = 16
NW = NC * NS


@functools.cache
def _build_gather_kernel():
    mesh = plsc.VectorSubcoreMesh(core_axis_name="c", subcore_axis_name="s")
    return functools.partial(
        pl.kernel,
        mesh=mesh,
        out_type=jax.ShapeDtypeStruct(
            (SEQ, HIDDEN // 8, BATCH // LANES, 8, LANES), jnp.float32
        ),
        scratch_types=[
            pltpu.VMEM((SEQ, LANES), jnp.int32),
            pltpu.VMEM((2, LANES), jnp.int32),
            pltpu.VMEM((2, LANES), jnp.int32),
            pltpu.VMEM((2, LANES, LANES), jnp.float32),
            pltpu.VMEM((2, 8, 8, LANES), jnp.float32),
            pltpu.SemaphoreType.DMA,
            pltpu.SemaphoreType.DMA,
            pltpu.SemaphoreType.DMA,
            pltpu.SemaphoreType.DMA,
        ],
        compiler_params=pltpu.CompilerParams(
            use_tc_tiling_on_sc=False, needs_layout_passes=False
        ),
    )(_gather_body)


def _gather_body(ids_hbm, table_hbm, out_hbm, ids_v, p_v, off_v, stage_v,
                 tile_v, sem_g0, sem_g1, sem_o0, sem_o1):
    wid = lax.axis_index("s") * NC + lax.axis_index("c")
    gsems = (sem_g0, sem_g1)
    osems = (sem_o0, sem_o1)

    pltpu.sync_copy(ids_hbm.at[:, pl.ds(wid * LANES, LANES)], ids_v)

    row16 = [lax.iota(jnp.int32, 16) + (16 * g) for g in range(8)]

    def prep(s, b):
        for g in range(8):
            v = ids_v[s, pl.ds(16 * g, 16)]
            p_v[b, pl.ds(16 * g, 16)] = lax.shift_right_logical(v, 1)
            off_v[b, pl.ds(16 * g, 16)] = lax.shift_left(v & 1, 6)
        pltpu.async_copy(table_hbm.at[p_v.at[b]], stage_v.at[b], gsems[b])

    def transpose_store(s, b):
        def hbody(hr, carry):
            for g in range(8):
                cols = off_v[b, pl.ds(16 * g, 16)] + hr * 8
                for r in range(8):
                    x = plsc.load_gather(stage_v.at[b], [row16[g], cols + r])
                    tile_v[b, hr, r, pl.ds(16 * g, 16)] = x
            return carry

        lax.fori_loop(0, 8, hbody, 0)
        pltpu.async_copy(tile_v.at[b], out_hbm.at[s, :, wid], osems[b])

    def wait_gather(b):
        pltpu.make_async_copy(
            table_hbm.at[pl.ds(0, LANES)], stage_v.at[b], gsems[b]
        ).wait()

    def wait_store(b):
        pltpu.make_async_copy(
            tile_v.at[b], out_hbm.at[0, :, wid], osems[b]
        ).wait()

    prep(0, 0)
    prep(1, 1)
    wait_gather(0)
    transpose_store(0, 0)
    wait_gather(1)
    prep(2, 0)
    transpose_store(1, 1)

    def body(t, carry):
        s = 2 * t + 1
        wait_gather(0)
        wait_store(0)
        prep(s, 1)
        transpose_store(s - 1, 0)
        wait_gather(1)
        wait_store(1)
        prep(s + 1, 0)
        transpose_store(s, 1)
        return carry

    lax.fori_loop(1, (SEQ - 2) // 2, body, 0)

    wait_gather(0)
    wait_store(0)
    prep(SEQ - 1, 1)
    transpose_store(SEQ - 2, 0)
    wait_gather(1)
    wait_store(1)
    transpose_store(SEQ - 1, 1)
    wait_store(0)
    wait_store(1)


def kernel(input_ids, W):
    ids_t = input_ids.T.astype(jnp.int32)
    table = W.reshape(VOCAB // 2, 2 * HIDDEN)
    out5 = _build_gather_kernel()(ids_t, table)
    return out5.transpose(2, 4, 0, 1, 3).reshape(BATCH, SEQ, HIDDEN)


# speedup vs baseline: 1.4162x; 1.4162x over previous
import functools

import jax
import jax.numpy as jnp
from jax import lax
from jax.experimental import pallas as pl
from jax.experimental.pallas import tpu as pltpu
from jax.experimental.pallas import tpu_sc as plsc

BATCH = 4096
SEQ = 200
HIDDEN = 64
VOCAB = 1000000
LANES = 128

NC = 2
NS = 16
NW = NC * NS


@functools.cache
def _build_gather_kernel():
    mesh = plsc.VectorSubcoreMesh(core_axis_name="c", subcore_axis_name="s")
    return functools.partial(
        pl.kernel,
        mesh=mesh,
        out_type=jax.ShapeDtypeStruct(
            (SEQ, HIDDEN // 8, BATCH // LANES, 8, LANES), jnp.float32
        ),
        scratch_types=[
            pltpu.VMEM((SEQ, LANES), jnp.int32),
            pltpu.VMEM((2, LANES), jnp.int32),
            pltpu.VMEM((2, LANES), jnp.int32),
            pltpu.VMEM((2, LANES, LANES), jnp.float32),
            pltpu.VMEM((2, 8, 8, LANES), jnp.float32),
            pltpu.SemaphoreType.DMA,
            pltpu.SemaphoreType.DMA,
            pltpu.SemaphoreType.DMA,
            pltpu.SemaphoreType.DMA,
        ],
        compiler_params=pltpu.CompilerParams(
            use_tc_tiling_on_sc=True, needs_layout_passes=False
        ),
    )(_gather_body)


def _gather_body(ids_hbm, table_hbm, out_hbm, ids_v, p_v, off_v, stage_v,
                 tile_v, sem_g0, sem_g1, sem_o0, sem_o1):
    wid = lax.axis_index("s") * NC + lax.axis_index("c")
    gsems = (sem_g0, sem_g1)
    osems = (sem_o0, sem_o1)

    pltpu.sync_copy(ids_hbm.at[:, pl.ds(wid * LANES, LANES)], ids_v)

    row16 = [lax.iota(jnp.int32, 16) + (16 * g) for g in range(8)]

    def prep(s, b):
        for g in range(8):
            v = ids_v[s, pl.ds(16 * g, 16)]
            p_v[b, pl.ds(16 * g, 16)] = lax.shift_right_logical(v, 1)
            off_v[b, pl.ds(16 * g, 16)] = lax.shift_left(v & 1, 6)
        pltpu.async_copy(table_hbm.at[p_v.at[b]], stage_v.at[b], gsems[b])

    def transpose_store(s, b):
        @plsc.parallel_loop(0, 8)
        def hbody(hr):
            for g in range(8):
                cols = off_v[b, pl.ds(16 * g, 16)] + hr * 8
                for r in range(8):
                    x = plsc.load_gather(stage_v.at[b], [row16[g], cols + r])
                    tile_v[b, hr, r, pl.ds(16 * g, 16)] = x
        pltpu.async_copy(tile_v.at[b], out_hbm.at[s, :, wid], osems[b])

    def wait_gather(b):
        pltpu.make_async_copy(
            table_hbm.at[pl.ds(0, LANES)], stage_v.at[b], gsems[b]
        ).wait()

    def wait_store(b):
        pltpu.make_async_copy(
            tile_v.at[b], out_hbm.at[0, :, wid], osems[b]
        ).wait()

    prep(0, 0)
    prep(1, 1)
    wait_gather(0)
    transpose_store(0, 0)
    wait_gather(1)
    prep(2, 0)
    transpose_store(1, 1)

    def body(t, carry):
        s = 2 * t + 1
        wait_gather(0)
        wait_store(0)
        prep(s, 1)
        transpose_store(s - 1, 0)
        wait_gather(1)
        wait_store(1)
        prep(s + 1, 0)
        transpose_store(s, 1)
        return carry

    lax.fori_loop(1, (SEQ - 2) // 2, body, 0)

    wait_gather(0)
    wait_store(0)
    prep(SEQ - 1, 1)
    transpose_store(SEQ - 2, 0)
    wait_gather(1)
    wait_store(1)
    transpose_store(SEQ - 1, 1)
    wait_store(0)
    wait_store(1)


def kernel(input_ids, W):
    ids_t = input_ids.T.astype(jnp.int32)
    table = W.reshape(VOCAB // 2, 2 * HIDDEN)
    out5 = _build_gather_kernel()(ids_t, table)
    return out5.transpose(2, 4, 0, 1, 3).reshape(BATCH, SEQ, HIDDEN)


# speedup vs baseline: 1.8696x; 1.3202x over previous
import functools

import jax
import jax.numpy as jnp
from jax import lax
from jax.experimental import pallas as pl
from jax.experimental.pallas import tpu as pltpu
from jax.experimental.pallas import tpu_sc as plsc

BATCH = 4096
SEQ = 200
HIDDEN = 64
VOCAB = 1000000
PADW = 128

NC = 2
NS = 16
NW = NC * NS

TOTAL = BATCH * SEQ
PER_W = TOTAL // NW
GRP = 128
K = 5
CHUNK = K * GRP
NCHUNK = PER_W // CHUNK


@functools.cache
def _build_gather_kernel():
    mesh = plsc.VectorSubcoreMesh(core_axis_name="c", subcore_axis_name="s")
    return functools.partial(
        pl.kernel,
        mesh=mesh,
        out_type=jax.ShapeDtypeStruct((TOTAL, PADW), jnp.float32),
        scratch_types=[
            pltpu.VMEM((K, GRP), jnp.int32),
            pltpu.VMEM((CHUNK, PADW), jnp.float32),
            pltpu.SemaphoreType.DMA,
        ],
        compiler_params=pltpu.CompilerParams(
            use_tc_tiling_on_sc=False, skip_device_barrier=True
        ),
    )(_gather_body)


def _gather_body(idx_hbm, table_hbm, out_hbm, idx_v, rows_v, sem):
    wid = lax.axis_index("s") * NC + lax.axis_index("c")
    base = wid * PER_W
    base_g = wid * (PER_W // GRP)

    def body(j, carry):
        off = base + j * CHUNK
        pltpu.sync_copy(idx_hbm.at[pl.ds(base_g + j * K, K)], idx_v)
        copies = []
        for g in range(K):
            copies.append(
                pltpu.async_copy(
                    table_hbm.at[idx_v.at[g]],
                    rows_v.at[pl.ds(g * GRP, GRP)],
                    sem,
                )
            )
        for c in copies:
            c.wait()
        pltpu.sync_copy(rows_v, out_hbm.at[pl.ds(off, CHUNK)])
        return carry

    lax.fori_loop(0, NCHUNK, body, 0)


def kernel(input_ids, W):
    idx = input_ids.reshape(TOTAL // GRP, GRP).astype(jnp.int32)
    Wp = jnp.pad(W, ((0, 0), (0, PADW - HIDDEN)))
    out = _build_gather_kernel()(idx, Wp)
    return out[:, :HIDDEN].reshape(BATCH, SEQ, HIDDEN)
